# Initial kernel scaffold; baseline (speedup 1.0000x reference)
import jax
import jax.numpy as jnp
from jax.experimental import pallas as pl


def kernel(embedding, emb_table, W1, b1, W2, b2):
    raise NotImplementedError("write your pallas kernel here")


# speedup vs baseline: 10.9152x; 10.9152x over previous
import numpy as np
import jax
import jax.numpy as jnp
from jax.experimental import pallas as pl
from jax.experimental.pallas import tpu as pltpu

_B = 8
_HIDDEN = 128
_EMB = 128
_MAX_DEPTH = 20
_NPD = 500
_MAX_PARENTS = 16
_TOTAL = 1 + _MAX_DEPTH * _NPD


def _dag_parent_indices() -> np.ndarray:
    rng = np.random.RandomState(0)
    parents = []
    next_idx = 2
    for _d in range(1, _MAX_DEPTH + 1):
        avail = next_idx - 1
        P = np.zeros((_NPD, _MAX_PARENTS), dtype=np.int64)
        for i in range(_NPD):
            k = min(int(rng.randint(1, _MAX_PARENTS + 1)), avail)
            ps = rng.choice(np.arange(1, next_idx, dtype=np.int64), size=k, replace=False)
            P[i, :k] = np.sort(ps)
        parents.append(P)
        next_idx += _NPD
    return np.stack(parents).astype(np.int32)


_IDX = _dag_parent_indices()


def _dag_kernel(idx_ref, emb_ref, embedding_ref, w1a_ref, w1b_ref, b1_ref,
                w2_ref, b2_ref, v_ref, pv_ref):
    d = pl.program_id(0)

    @pl.when(d == 0)
    def _init():
        v_ref[0] = jnp.zeros((_B, _HIDDEN), jnp.float32)
        v_ref[1] = embedding_ref[...]

    def gather_body(i, carry):
        vs = [v_ref[idx_ref[0, i, k]] for k in range(_MAX_PARENTS)]
        while len(vs) > 1:
            vs = [vs[a] + vs[a + 1] for a in range(0, len(vs), 2)]
        pv_ref[i] = vs[0]
        return carry

    jax.lax.fori_loop(0, _NPD, gather_body, 0)

    pv = pv_ref[...].reshape(_NPD * _B, _HIDDEN)
    emb = emb_ref[0]
    e1 = jnp.dot(emb, w1b_ref[...], preferred_element_type=jnp.float32)
    e1 = jnp.broadcast_to(e1[:, None, :], (_NPD, _B, _HIDDEN))
    e1 = e1.reshape(_NPD * _B, _HIDDEN)
    h = jnp.maximum(
        jnp.dot(pv, w1a_ref[...], preferred_element_type=jnp.float32)
        + e1 + b1_ref[...], 0.0)
    y = (jnp.dot(h, w2_ref[...], preferred_element_type=jnp.float32)
         + b2_ref[...] + pv)
    base = 2 + d * _NPD
    v_ref[pl.ds(base, _NPD)] = y.reshape(_NPD, _B, _HIDDEN)


def kernel(embedding, emb_table, W1, b1, W2, b2):
    idx = jnp.asarray(_IDX)
    emb_sl = jax.lax.slice(emb_table, (2, 0), (_TOTAL + 1, _EMB))
    emb_sl = emb_sl.reshape(_MAX_DEPTH, _NPD, _EMB)
    w1a = W1[:_HIDDEN]
    w1b = W1[_HIDDEN:]
    b1r = b1.reshape(1, _HIDDEN)
    b2r = b2.reshape(1, _HIDDEN)

    out = pl.pallas_call(
        _dag_kernel,
        grid=(_MAX_DEPTH,),
        in_specs=[
            pl.BlockSpec((1, _NPD, _MAX_PARENTS), lambda d: (d, 0, 0),
                         memory_space=pltpu.SMEM),
            pl.BlockSpec((1, _NPD, _EMB), lambda d: (d, 0, 0)),
            pl.BlockSpec((_B, _HIDDEN), lambda d: (0, 0)),
            pl.BlockSpec((_HIDDEN, _HIDDEN), lambda d: (0, 0)),
            pl.BlockSpec((_EMB, _HIDDEN), lambda d: (0, 0)),
            pl.BlockSpec((1, _HIDDEN), lambda d: (0, 0)),
            pl.BlockSpec((_HIDDEN, _HIDDEN), lambda d: (0, 0)),
            pl.BlockSpec((1, _HIDDEN), lambda d: (0, 0)),
        ],
        out_specs=pl.BlockSpec((_TOTAL + 1, _B, _HIDDEN), lambda d: (0, 0, 0)),
        out_shape=jax.ShapeDtypeStruct((_TOTAL + 1, _B, _HIDDEN), jnp.float32),
        scratch_shapes=[pltpu.VMEM((_NPD, _B, _HIDDEN), jnp.float32)],
        compiler_params=pltpu.CompilerParams(
            dimension_semantics=("arbitrary",),
            vmem_limit_bytes=56 * 1024 * 1024,
        ),
    )(idx, emb_sl, embedding, w1a, w1b, b1r, W2, b2r)
    return jnp.transpose(out[1:], (1, 0, 2))


# speedup vs baseline: 17.3774x; 1.5920x over previous
import numpy as np
import jax
import jax.numpy as jnp
from jax.experimental import pallas as pl
from jax.experimental.pallas import tpu as pltpu

_B = 8
_HIDDEN = 128
_EMB = 128
_MAX_DEPTH = 20
_NPD = 500
_MAX_PARENTS = 16
_TOTAL = 1 + _MAX_DEPTH * _NPD


def _dag_parent_indices() -> np.ndarray:
    rng = np.random.RandomState(0)
    parents = []
    next_idx = 2
    for _d in range(1, _MAX_DEPTH + 1):
        avail = next_idx - 1
        P = np.zeros((_NPD, _MAX_PARENTS), dtype=np.int64)
        for i in range(_NPD):
            k = min(int(rng.randint(1, _MAX_PARENTS + 1)), avail)
            ps = rng.choice(np.arange(1, next_idx, dtype=np.int64), size=k, replace=False)
            P[i, :k] = np.sort(ps)
        parents.append(P)
        next_idx += _NPD
    return np.stack(parents).astype(np.int32)


_IDX = _dag_parent_indices()

_PV_ROWS = 504
_DUMMY_DST = 500


def _build_streams():
    streams, metas = [], []
    for d in range(_MAX_DEPTH):
        buckets = [[] for _ in range(_MAX_PARENTS + 1)]
        P = _IDX[d]
        for i in range(_NPD):
            k = int((P[i] != 0).sum())
            buckets[k].append([i] + P[i, :k].tolist())
        flat, n_pairs, offs = [], [], []
        for k in range(1, _MAX_PARENTS + 1):
            recs = buckets[k]
            if len(recs) % 2:
                recs = recs + [[_DUMMY_DST] + [0] * k]
            offs.append(len(flat))
            n_pairs.append(len(recs) // 2)
            for r in recs:
                flat.extend(r)
        metas.append(n_pairs + offs)
        streams.append(flat)
    maxlen = max(len(s) for s in streams)
    stream = np.zeros((_MAX_DEPTH, 1, maxlen), dtype=np.int32)
    for d, s in enumerate(streams):
        stream[d, 0, :len(s)] = s
    return stream, np.asarray(metas, dtype=np.int32).reshape(_MAX_DEPTH, 1, -1)


_STREAM, _META = _build_streams()


def _dag_kernel(meta_ref, stream_ref, emb_ref, embedding_ref, w1a_ref,
                w1b_ref, b1_ref, w2_ref, b2_ref, v_ref, pv_ref):
    d = pl.program_id(0)

    @pl.when(d == 0)
    def _init():
        v_ref[0] = jnp.zeros((_B, _HIDDEN), jnp.float32)
        v_ref[1] = embedding_ref[...]

    def _one(base, k):
        dst = stream_ref[0, 0, base]
        vs = [v_ref[stream_ref[0, 0, base + 1 + j]] for j in range(k)]
        while len(vs) > 1:
            nxt = [vs[a] + vs[a + 1] for a in range(0, len(vs) - 1, 2)]
            if len(vs) % 2:
                nxt.append(vs[-1])
            vs = nxt
        pv_ref[dst] = vs[0]

    for k in range(1, _MAX_PARENTS + 1):
        n_pairs = meta_ref[0, 0, k - 1]
        off = meta_ref[0, 0, _MAX_PARENTS + k - 1]
        stride = k + 1

        def pair_body(i, carry, k=k, off=off, stride=stride):
            base = off + i * (2 * stride)
            _one(base, k)
            _one(base + stride, k)
            return carry

        jax.lax.fori_loop(0, n_pairs, pair_body, 0)

    pv = pv_ref[pl.ds(0, _NPD)].reshape(_NPD * _B, _HIDDEN)
    emb = emb_ref[0]
    e1 = jnp.dot(emb, w1b_ref[...], preferred_element_type=jnp.float32)
    e1 = jnp.broadcast_to(e1[:, None, :], (_NPD, _B, _HIDDEN))
    e1 = e1.reshape(_NPD * _B, _HIDDEN)
    h = jnp.maximum(
        jnp.dot(pv, w1a_ref[...], preferred_element_type=jnp.float32)
        + e1 + b1_ref[...], 0.0)
    y = (jnp.dot(h, w2_ref[...], preferred_element_type=jnp.float32)
         + b2_ref[...] + pv)
    base = 2 + d * _NPD
    v_ref[pl.ds(base, _NPD)] = y.reshape(_NPD, _B, _HIDDEN)


def kernel(embedding, emb_table, W1, b1, W2, b2):
    meta = jnp.asarray(_META)
    stream = jnp.asarray(_STREAM)
    emb_sl = jax.lax.slice(emb_table, (2, 0), (_TOTAL + 1, _EMB))
    emb_sl = emb_sl.reshape(_MAX_DEPTH, _NPD, _EMB)
    w1a = W1[:_HIDDEN]
    w1b = W1[_HIDDEN:]
    b1r = b1.reshape(1, _HIDDEN)
    b2r = b2.reshape(1, _HIDDEN)

    out = pl.pallas_call(
        _dag_kernel,
        grid=(_MAX_DEPTH,),
        in_specs=[
            pl.BlockSpec((1, 1, 2 * _MAX_PARENTS), lambda d: (d, 0, 0),
                         memory_space=pltpu.SMEM),
            pl.BlockSpec((1, 1, _STREAM.shape[2]), lambda d: (d, 0, 0),
                         memory_space=pltpu.SMEM),
            pl.BlockSpec((1, _NPD, _EMB), lambda d: (d, 0, 0)),
            pl.BlockSpec((_B, _HIDDEN), lambda d: (0, 0)),
            pl.BlockSpec((_HIDDEN, _HIDDEN), lambda d: (0, 0)),
            pl.BlockSpec((_EMB, _HIDDEN), lambda d: (0, 0)),
            pl.BlockSpec((1, _HIDDEN), lambda d: (0, 0)),
            pl.BlockSpec((_HIDDEN, _HIDDEN), lambda d: (0, 0)),
            pl.BlockSpec((1, _HIDDEN), lambda d: (0, 0)),
        ],
        out_specs=pl.BlockSpec((_TOTAL + 1, _B, _HIDDEN), lambda d: (0, 0, 0)),
        out_shape=jax.ShapeDtypeStruct((_TOTAL + 1, _B, _HIDDEN), jnp.float32),
        scratch_shapes=[pltpu.VMEM((_PV_ROWS, _B, _HIDDEN), jnp.float32)],
        compiler_params=pltpu.CompilerParams(
            dimension_semantics=("arbitrary",),
            vmem_limit_bytes=56 * 1024 * 1024,
        ),
    )(meta, stream, emb_sl, embedding, w1a, w1b, b1r, W2, b2r)
    return jnp.transpose(out[1:], (1, 0, 2))


# speedup vs baseline: 18.3455x; 1.0557x over previous
import numpy as np
import jax
import jax.numpy as jnp
from jax.experimental import pallas as pl
from jax.experimental.pallas import tpu as pltpu

_B = 8
_HIDDEN = 128
_EMB = 128
_MAX_DEPTH = 20
_NPD = 500
_MAX_PARENTS = 16
_TOTAL = 1 + _MAX_DEPTH * _NPD


def _dag_parent_indices() -> np.ndarray:
    rng = np.random.RandomState(0)
    parents = []
    next_idx = 2
    for _d in range(1, _MAX_DEPTH + 1):
        avail = next_idx - 1
        P = np.zeros((_NPD, _MAX_PARENTS), dtype=np.int64)
        for i in range(_NPD):
            k = min(int(rng.randint(1, _MAX_PARENTS + 1)), avail)
            ps = rng.choice(np.arange(1, next_idx, dtype=np.int64), size=k, replace=False)
            P[i, :k] = np.sort(ps)
        parents.append(P)
        next_idx += _NPD
    return np.stack(parents).astype(np.int32)


_IDX = _dag_parent_indices()

_PV_ROWS = 504
_DUMMY_DST = 500


def _unroll_for(k: int) -> int:
    return 4 if k <= 8 else 2


def _build_streams():
    streams, metas = [], []
    for d in range(_MAX_DEPTH):
        buckets = [[] for _ in range(_MAX_PARENTS + 1)]
        P = _IDX[d]
        if d > 0:
            for i in range(_NPD):
                k = int((P[i] != 0).sum())
                buckets[k].append([i] + P[i, :k].tolist())
        flat, n_iters, offs = [], [], []
        for k in range(1, _MAX_PARENTS + 1):
            u = _unroll_for(k)
            recs = buckets[k]
            while len(recs) % u:
                recs = recs + [[_DUMMY_DST] + [0] * k]
            offs.append(len(flat))
            n_iters.append(len(recs) // u)
            for r in recs:
                flat.extend(r)
        metas.append(n_iters + offs)
        streams.append(flat)
    maxlen = max(len(s) for s in streams)
    stream = np.zeros((_MAX_DEPTH, 1, maxlen), dtype=np.int32)
    for d, s in enumerate(streams):
        stream[d, 0, :len(s)] = s
    return stream, np.asarray(metas, dtype=np.int32).reshape(_MAX_DEPTH, 1, -1)


_STREAM, _META = _build_streams()


def _dag_kernel(meta_ref, stream_ref, emb_ref, embedding_ref, w1a_ref,
                w1b_ref, b1_ref, w2_ref, b2_ref, v_ref, pv_ref):
    d = pl.program_id(0)

    @pl.when(d == 0)
    def _init():
        v_ref[0] = jnp.zeros((_B, _HIDDEN), jnp.float32)
        v_ref[1] = embedding_ref[...]
        pv_ref[pl.ds(0, _NPD)] = jnp.broadcast_to(
            embedding_ref[...], (_NPD, _B, _HIDDEN))

    def _one(base, k):
        dst = stream_ref[0, 0, base]
        vs = [v_ref[stream_ref[0, 0, base + 1 + j]] for j in range(k)]
        while len(vs) > 1:
            nxt = [vs[a] + vs[a + 1] for a in range(0, len(vs) - 1, 2)]
            if len(vs) % 2:
                nxt.append(vs[-1])
            vs = nxt
        pv_ref[dst] = vs[0]

    for k in range(1, _MAX_PARENTS + 1):
        n_iters = meta_ref[0, 0, k - 1]
        off = meta_ref[0, 0, _MAX_PARENTS + k - 1]
        stride = k + 1
        u = _unroll_for(k)

        def body(i, carry, k=k, off=off, stride=stride, u=u):
            base = off + i * (u * stride)
            for r in range(u):
                _one(base + r * stride, k)
            return carry

        jax.lax.fori_loop(0, n_iters, body, 0)

    pv = pv_ref[pl.ds(0, _NPD)].reshape(_NPD * _B, _HIDDEN)
    emb = emb_ref[0]
    e1 = jnp.dot(emb, w1b_ref[...], preferred_element_type=jnp.float32)
    e1 = jnp.broadcast_to(e1[:, None, :], (_NPD, _B, _HIDDEN))
    e1 = e1.reshape(_NPD * _B, _HIDDEN)
    h = jnp.maximum(
        jnp.dot(pv, w1a_ref[...], preferred_element_type=jnp.float32)
        + e1 + b1_ref[...], 0.0)
    y = (jnp.dot(h, w2_ref[...], preferred_element_type=jnp.float32)
         + b2_ref[...] + pv)
    base = 2 + d * _NPD
    v_ref[pl.ds(base, _NPD)] = y.reshape(_NPD, _B, _HIDDEN)


def kernel(embedding, emb_table, W1, b1, W2, b2):
    meta = jnp.asarray(_META)
    stream = jnp.asarray(_STREAM)
    emb_sl = jax.lax.slice(emb_table, (2, 0), (_TOTAL + 1, _EMB))
    emb_sl = emb_sl.reshape(_MAX_DEPTH, _NPD, _EMB)
    w1a = W1[:_HIDDEN]
    w1b = W1[_HIDDEN:]
    b1r = b1.reshape(1, _HIDDEN)
    b2r = b2.reshape(1, _HIDDEN)

    out = pl.pallas_call(
        _dag_kernel,
        grid=(_MAX_DEPTH,),
        in_specs=[
            pl.BlockSpec((1, 1, 2 * _MAX_PARENTS), lambda d: (d, 0, 0),
                         memory_space=pltpu.SMEM),
            pl.BlockSpec((1, 1, _STREAM.shape[2]), lambda d: (d, 0, 0),
                         memory_space=pltpu.SMEM),
            pl.BlockSpec((1, _NPD, _EMB), lambda d: (d, 0, 0)),
            pl.BlockSpec((_B, _HIDDEN), lambda d: (0, 0)),
            pl.BlockSpec((_HIDDEN, _HIDDEN), lambda d: (0, 0)),
            pl.BlockSpec((_EMB, _HIDDEN), lambda d: (0, 0)),
            pl.BlockSpec((1, _HIDDEN), lambda d: (0, 0)),
            pl.BlockSpec((_HIDDEN, _HIDDEN), lambda d: (0, 0)),
            pl.BlockSpec((1, _HIDDEN), lambda d: (0, 0)),
        ],
        out_specs=pl.BlockSpec((_TOTAL + 1, _B, _HIDDEN), lambda d: (0, 0, 0)),
        out_shape=jax.ShapeDtypeStruct((_TOTAL + 1, _B, _HIDDEN), jnp.float32),
        scratch_shapes=[pltpu.VMEM((_PV_ROWS, _B, _HIDDEN), jnp.float32)],
        compiler_params=pltpu.CompilerParams(
            dimension_semantics=("arbitrary",),
            vmem_limit_bytes=56 * 1024 * 1024,
        ),
    )(meta, stream, emb_sl, embedding, w1a, w1b, b1r, W2, b2r)
    return jnp.transpose(out[1:], (1, 0, 2))


# speedup vs baseline: 19.0112x; 1.0363x over previous
import numpy as np
import jax
import jax.numpy as jnp
from jax.experimental import pallas as pl
from jax.experimental.pallas import tpu as pltpu

_B = 8
_HIDDEN = 128
_EMB = 128
_MAX_DEPTH = 20
_NPD = 500
_MAX_PARENTS = 16
_TOTAL = 1 + _MAX_DEPTH * _NPD


def _dag_parent_indices() -> np.ndarray:
    rng = np.random.RandomState(0)
    parents = []
    next_idx = 2
    for _d in range(1, _MAX_DEPTH + 1):
        avail = next_idx - 1
        P = np.zeros((_NPD, _MAX_PARENTS), dtype=np.int64)
        for i in range(_NPD):
            k = min(int(rng.randint(1, _MAX_PARENTS + 1)), avail)
            ps = rng.choice(np.arange(1, next_idx, dtype=np.int64), size=k, replace=False)
            P[i, :k] = np.sort(ps)
        parents.append(P)
        next_idx += _NPD
    return np.stack(parents).astype(np.int32)


_IDX = _dag_parent_indices()

_PV_ROWS = 504
_DUMMY_DST = 500


def _unroll_for(k: int) -> int:
    if k <= 3:
        return 16
    if k <= 6:
        return 8
    return 4


def _build_streams():
    streams, metas = [], []
    for d in range(_MAX_DEPTH):
        buckets = [[] for _ in range(_MAX_PARENTS + 1)]
        P = _IDX[d]
        if d > 0:
            for i in range(_NPD):
                k = int((P[i] != 0).sum())
                buckets[k].append([i] + P[i, :k].tolist())
        flat, n_iters, offs = [], [], []
        for k in range(1, _MAX_PARENTS + 1):
            u = _unroll_for(k)
            recs = buckets[k]
            while len(recs) % u:
                recs = recs + [[_DUMMY_DST] + [0] * k]
            offs.append(len(flat))
            n_iters.append(len(recs) // u)
            for r in recs:
                flat.extend(r)
        metas.append(n_iters + offs)
        streams.append(flat)
    maxlen = max(len(s) for s in streams)
    stream = np.zeros((_MAX_DEPTH, 1, maxlen), dtype=np.int32)
    for d, s in enumerate(streams):
        stream[d, 0, :len(s)] = s
    return stream, np.asarray(metas, dtype=np.int32).reshape(_MAX_DEPTH, 1, -1)


_STREAM, _META = _build_streams()


def _dag_kernel(meta_ref, stream_ref, emb_ref, embedding_ref, w1a_ref,
                w1b_ref, b1_ref, w2_ref, b2_ref, v_ref, pv_ref):
    d = pl.program_id(0)

    @pl.when(d == 0)
    def _init():
        v_ref[0] = jnp.zeros((_B, _HIDDEN), jnp.float32)
        v_ref[1] = embedding_ref[...]
        t = jnp.dot(embedding_ref[...], w1a_ref[...],
                    preferred_element_type=jnp.float32)
        e1 = jnp.dot(emb_ref[0], w1b_ref[...],
                     preferred_element_type=jnp.float32)
        h = jnp.maximum(
            e1[:, None, :] + (t + b1_ref[...])[None, :, :], 0.0)
        h = h.reshape(_NPD * _B, _HIDDEN)
        y = (jnp.dot(h, w2_ref[...], preferred_element_type=jnp.float32)
             + b2_ref[...]).reshape(_NPD, _B, _HIDDEN) + embedding_ref[...]
        v_ref[pl.ds(2, _NPD)] = y

    @pl.when(d > 0)
    def _step():
        def _one(base, k):
            dst = stream_ref[0, 0, base]
            vs = [v_ref[stream_ref[0, 0, base + 1 + j]] for j in range(k)]
            while len(vs) > 1:
                nxt = [vs[a] + vs[a + 1] for a in range(0, len(vs) - 1, 2)]
                if len(vs) % 2:
                    nxt.append(vs[-1])
                vs = nxt
            pv_ref[dst] = vs[0]

        for k in range(1, _MAX_PARENTS + 1):
            n_iters = meta_ref[0, 0, k - 1]
            off = meta_ref[0, 0, _MAX_PARENTS + k - 1]
            stride = k + 1
            u = _unroll_for(k)

            def body(i, carry, k=k, off=off, stride=stride, u=u):
                base = off + i * (u * stride)
                for r in range(u):
                    _one(base + r * stride, k)
                return carry

            jax.lax.fori_loop(0, n_iters, body, 0)

        pv = pv_ref[pl.ds(0, _NPD)].reshape(_NPD * _B, _HIDDEN)
        emb = emb_ref[0]
        e1 = jnp.dot(emb, w1b_ref[...], preferred_element_type=jnp.float32)
        e1 = jnp.broadcast_to(e1[:, None, :], (_NPD, _B, _HIDDEN))
        e1 = e1.reshape(_NPD * _B, _HIDDEN)
        h = jnp.maximum(
            jnp.dot(pv, w1a_ref[...], preferred_element_type=jnp.float32)
            + e1 + b1_ref[...], 0.0)
        y = (jnp.dot(h, w2_ref[...], preferred_element_type=jnp.float32)
             + b2_ref[...] + pv)
        base = 2 + d * _NPD
        v_ref[pl.ds(base, _NPD)] = y.reshape(_NPD, _B, _HIDDEN)


def kernel(embedding, emb_table, W1, b1, W2, b2):
    meta = jnp.asarray(_META)
    stream = jnp.asarray(_STREAM)
    emb_sl = jax.lax.slice(emb_table, (2, 0), (_TOTAL + 1, _EMB))
    emb_sl = emb_sl.reshape(_MAX_DEPTH, _NPD, _EMB)
    w1a = W1[:_HIDDEN]
    w1b = W1[_HIDDEN:]
    b1r = b1.reshape(1, _HIDDEN)
    b2r = b2.reshape(1, _HIDDEN)

    out = pl.pallas_call(
        _dag_kernel,
        grid=(_MAX_DEPTH,),
        in_specs=[
            pl.BlockSpec((1, 1, 2 * _MAX_PARENTS), lambda d: (d, 0, 0),
                         memory_space=pltpu.SMEM),
            pl.BlockSpec((1, 1, _STREAM.shape[2]), lambda d: (d, 0, 0),
                         memory_space=pltpu.SMEM),
            pl.BlockSpec((1, _NPD, _EMB), lambda d: (d, 0, 0)),
            pl.BlockSpec((_B, _HIDDEN), lambda d: (0, 0)),
            pl.BlockSpec((_HIDDEN, _HIDDEN), lambda d: (0, 0)),
            pl.BlockSpec((_EMB, _HIDDEN), lambda d: (0, 0)),
            pl.BlockSpec((1, _HIDDEN), lambda d: (0, 0)),
            pl.BlockSpec((_HIDDEN, _HIDDEN), lambda d: (0, 0)),
            pl.BlockSpec((1, _HIDDEN), lambda d: (0, 0)),
        ],
        out_specs=pl.BlockSpec((_TOTAL + 1, _B, _HIDDEN), lambda d: (0, 0, 0)),
        out_shape=jax.ShapeDtypeStruct((_TOTAL + 1, _B, _HIDDEN), jnp.float32),
        scratch_shapes=[pltpu.VMEM((_PV_ROWS, _B, _HIDDEN), jnp.float32)],
        compiler_params=pltpu.CompilerParams(
            dimension_semantics=("arbitrary",),
            vmem_limit_bytes=56 * 1024 * 1024,
        ),
    )(meta, stream, emb_sl, embedding, w1a, w1b, b1r, W2, b2r)
    return jnp.transpose(out[1:], (1, 0, 2))


# speedup vs baseline: 21.9312x; 1.1536x over previous
import numpy as np
import jax
import jax.numpy as jnp
from jax.experimental import pallas as pl
from jax.experimental.pallas import tpu as pltpu

_B = 8
_HIDDEN = 128
_EMB = 128
_MAX_DEPTH = 20
_NPD = 500
_MAX_PARENTS = 16
_TOTAL = 1 + _MAX_DEPTH * _NPD


def _dag_parent_indices() -> np.ndarray:
    rng = np.random.RandomState(0)
    parents = []
    next_idx = 2
    for _d in range(1, _MAX_DEPTH + 1):
        avail = next_idx - 1
        P = np.zeros((_NPD, _MAX_PARENTS), dtype=np.int64)
        for i in range(_NPD):
            k = min(int(rng.randint(1, _MAX_PARENTS + 1)), avail)
            ps = rng.choice(np.arange(1, next_idx, dtype=np.int64), size=k, replace=False)
            P[i, :k] = np.sort(ps)
        parents.append(P)
        next_idx += _NPD
    return np.stack(parents).astype(np.int32)


_IDX = _dag_parent_indices()

_PV_ROWS = 504
_DUMMY_DST = 500
_RS = _B


def _unroll_for(k: int) -> int:
    if k <= 3:
        return 16
    if k <= 6:
        return 8
    return 4


def _build_streams():
    streams, metas = [], []
    for d in range(_MAX_DEPTH):
        buckets = [[] for _ in range(_MAX_PARENTS + 1)]
        P = _IDX[d]
        if d > 0:
            for i in range(_NPD):
                k = int((P[i] != 0).sum())
                buckets[k].append([i * _RS] + (P[i, :k] * _RS).tolist())
        flat, n_iters, offs = [], [], []
        for k in range(1, _MAX_PARENTS + 1):
            u = _unroll_for(k)
            recs = buckets[k]
            while len(recs) % u:
                recs = recs + [[_DUMMY_DST * _RS] + [0] * k]
            offs.append(len(flat))
            n_iters.append(len(recs) // u)
            for r in recs:
                flat.extend(r)
        metas.append(n_iters + offs)
        streams.append(flat)
    maxlen = max(len(s) for s in streams)
    stream = np.zeros((_MAX_DEPTH, 1, maxlen), dtype=np.int32)
    for d, s in enumerate(streams):
        stream[d, 0, :len(s)] = s
    return stream, np.asarray(metas, dtype=np.int32).reshape(_MAX_DEPTH, 1, -1)


_STREAM, _META = _build_streams()


def _dag_kernel(meta_ref, stream_ref, emb_ref, embedding_ref, w1a_ref,
                w1b_ref, b1_ref, w2_ref, b2_ref, v_ref, pv_ref):
    d = pl.program_id(0)

    @pl.when(d == 0)
    def _init():
        v_ref[pl.ds(0, _RS)] = jnp.zeros((_RS, _HIDDEN), jnp.float32)
        v_ref[pl.ds(_RS, _RS)] = embedding_ref[...]
        t = jnp.dot(embedding_ref[...], w1a_ref[...],
                    preferred_element_type=jnp.float32)
        e1 = jnp.dot(emb_ref[0], w1b_ref[...],
                     preferred_element_type=jnp.float32)
        h = jnp.maximum(
            e1[:, None, :] + (t + b1_ref[...])[None, :, :], 0.0)
        h = h.reshape(_NPD * _B, _HIDDEN)
        y = (jnp.dot(h, w2_ref[...], preferred_element_type=jnp.float32)
             + b2_ref[...]).reshape(_NPD, _B, _HIDDEN) + embedding_ref[...]
        v_ref[pl.ds(2 * _RS, _NPD * _RS)] = y.reshape(_NPD * _RS, _HIDDEN)

    @pl.when(d > 0)
    def _step():
        def _one(base, k):
            dst = stream_ref[0, 0, base]
            vs = [v_ref[pl.ds(stream_ref[0, 0, base + 1 + j], _RS)]
                  for j in range(k)]
            while len(vs) > 1:
                nxt = [vs[a] + vs[a + 1] for a in range(0, len(vs) - 1, 2)]
                if len(vs) % 2:
                    nxt.append(vs[-1])
                vs = nxt
            pv_ref[pl.ds(dst, _RS)] = vs[0]

        for k in range(1, _MAX_PARENTS + 1):
            n_iters = meta_ref[0, 0, k - 1]
            off = meta_ref[0, 0, _MAX_PARENTS + k - 1]
            stride = k + 1
            u = _unroll_for(k)

            def body(i, carry, k=k, off=off, stride=stride, u=u):
                base = off + i * (u * stride)
                for r in range(u):
                    _one(base + r * stride, k)
                return carry

            jax.lax.fori_loop(0, n_iters, body, 0)

        pv = pv_ref[pl.ds(0, _NPD * _RS)]
        emb = emb_ref[0]
        e1 = jnp.dot(emb, w1b_ref[...], preferred_element_type=jnp.float32)
        e1 = jnp.broadcast_to(e1[:, None, :], (_NPD, _B, _HIDDEN))
        e1 = e1.reshape(_NPD * _B, _HIDDEN)
        h = jnp.maximum(
            jnp.dot(pv, w1a_ref[...], preferred_element_type=jnp.float32)
            + e1 + b1_ref[...], 0.0)
        y = (jnp.dot(h, w2_ref[...], preferred_element_type=jnp.float32)
             + b2_ref[...] + pv)
        base = (2 + d * _NPD) * _RS
        v_ref[pl.ds(base, _NPD * _RS)] = y


def kernel(embedding, emb_table, W1, b1, W2, b2):
    meta = jnp.asarray(_META)
    stream = jnp.asarray(_STREAM)
    emb_sl = jax.lax.slice(emb_table, (2, 0), (_TOTAL + 1, _EMB))
    emb_sl = emb_sl.reshape(_MAX_DEPTH, _NPD, _EMB)
    w1a = W1[:_HIDDEN]
    w1b = W1[_HIDDEN:]
    b1r = b1.reshape(1, _HIDDEN)
    b2r = b2.reshape(1, _HIDDEN)

    out = pl.pallas_call(
        _dag_kernel,
        grid=(_MAX_DEPTH,),
        in_specs=[
            pl.BlockSpec((1, 1, 2 * _MAX_PARENTS), lambda d: (d, 0, 0),
                         memory_space=pltpu.SMEM),
            pl.BlockSpec((1, 1, _STREAM.shape[2]), lambda d: (d, 0, 0),
                         memory_space=pltpu.SMEM),
            pl.BlockSpec((1, _NPD, _EMB), lambda d: (d, 0, 0)),
            pl.BlockSpec((_B, _HIDDEN), lambda d: (0, 0)),
            pl.BlockSpec((_HIDDEN, _HIDDEN), lambda d: (0, 0)),
            pl.BlockSpec((_EMB, _HIDDEN), lambda d: (0, 0)),
            pl.BlockSpec((1, _HIDDEN), lambda d: (0, 0)),
            pl.BlockSpec((_HIDDEN, _HIDDEN), lambda d: (0, 0)),
            pl.BlockSpec((1, _HIDDEN), lambda d: (0, 0)),
        ],
        out_specs=pl.BlockSpec(((_TOTAL + 1) * _RS, _HIDDEN), lambda d: (0, 0)),
        out_shape=jax.ShapeDtypeStruct(((_TOTAL + 1) * _RS, _HIDDEN),
                                       jnp.float32),
        scratch_shapes=[pltpu.VMEM((_PV_ROWS * _RS, _HIDDEN), jnp.float32)],
        compiler_params=pltpu.CompilerParams(
            dimension_semantics=("arbitrary",),
            vmem_limit_bytes=56 * 1024 * 1024,
        ),
    )(meta, stream, emb_sl, embedding, w1a, w1b, b1r, W2, b2r)
    out = out.reshape(_TOTAL + 1, _B, _HIDDEN)
    return jnp.transpose(out[1:], (1, 0, 2))


# speedup vs baseline: 22.0315x; 1.0046x over previous
import numpy as np
import jax
import jax.numpy as jnp
from jax.experimental import pallas as pl
from jax.experimental.pallas import tpu as pltpu

_B = 8
_HIDDEN = 128
_EMB = 128
_MAX_DEPTH = 20
_NPD = 500
_MAX_PARENTS = 16
_TOTAL = 1 + _MAX_DEPTH * _NPD


def _dag_parent_indices() -> np.ndarray:
    rng = np.random.RandomState(0)
    parents = []
    next_idx = 2
    for _d in range(1, _MAX_DEPTH + 1):
        avail = next_idx - 1
        P = np.zeros((_NPD, _MAX_PARENTS), dtype=np.int64)
        for i in range(_NPD):
            k = min(int(rng.randint(1, _MAX_PARENTS + 1)), avail)
            ps = rng.choice(np.arange(1, next_idx, dtype=np.int64), size=k, replace=False)
            P[i, :k] = np.sort(ps)
        parents.append(P)
        next_idx += _NPD
    return np.stack(parents).astype(np.int32)


_IDX = _dag_parent_indices()

_PV_ROWS = 504
_DUMMY_DST = 500
_RS = _B


def _unroll_for(k: int) -> int:
    if k <= 3:
        return 16
    if k <= 8:
        return 8
    return 4


def _build_streams():
    streams, metas = [], []
    for d in range(_MAX_DEPTH):
        buckets = [[] for _ in range(_MAX_PARENTS + 1)]
        P = _IDX[d]
        if d > 0:
            for i in range(_NPD):
                k = int((P[i] != 0).sum())
                buckets[k].append([i * _RS] + (P[i, :k] * _RS).tolist())
        flat, n_iters, offs = [], [], []
        for k in range(1, _MAX_PARENTS + 1):
            u = _unroll_for(k)
            recs = buckets[k]
            while len(recs) % u:
                recs = recs + [[_DUMMY_DST * _RS] + [0] * k]
            offs.append(len(flat))
            n_iters.append(len(recs) // u)
            for r in recs:
                flat.extend(r)
        metas.append(n_iters + offs)
        streams.append(flat)
    maxlen = max(len(s) for s in streams)
    stream = np.zeros((_MAX_DEPTH, 1, maxlen), dtype=np.int32)
    for d, s in enumerate(streams):
        stream[d, 0, :len(s)] = s
    return stream, np.asarray(metas, dtype=np.int32).reshape(_MAX_DEPTH, 1, -1)


_STREAM, _META = _build_streams()


def _dag_kernel(meta_ref, stream_ref, emb_ref, embedding_ref, w1a_ref,
                w1b_ref, b1_ref, w2_ref, b2_ref, v_ref, pv_ref):
    d = pl.program_id(0)

    @pl.when(d == 0)
    def _init():
        v_ref[pl.ds(0, _RS)] = jnp.zeros((_RS, _HIDDEN), jnp.float32)
        v_ref[pl.ds(_RS, _RS)] = embedding_ref[...]
        t = jnp.dot(embedding_ref[...], w1a_ref[...],
                    preferred_element_type=jnp.float32)
        e1 = jnp.dot(emb_ref[0], w1b_ref[...],
                     preferred_element_type=jnp.float32)
        h = jnp.maximum(
            e1[:, None, :] + (t + b1_ref[...])[None, :, :], 0.0)
        h = h.reshape(_NPD * _B, _HIDDEN)
        y = (jnp.dot(h, w2_ref[...], preferred_element_type=jnp.float32)
             + b2_ref[...]).reshape(_NPD, _B, _HIDDEN) + embedding_ref[...]
        v_ref[pl.ds(2 * _RS, _NPD * _RS)] = y.reshape(_NPD * _RS, _HIDDEN)

    @pl.when(d > 0)
    def _step():
        def _one(base, k):
            dst = stream_ref[0, 0, base]
            vs = [v_ref[pl.ds(stream_ref[0, 0, base + 1 + j], _RS)]
                  for j in range(k)]
            while len(vs) > 1:
                nxt = [vs[a] + vs[a + 1] for a in range(0, len(vs) - 1, 2)]
                if len(vs) % 2:
                    nxt.append(vs[-1])
                vs = nxt
            pv_ref[pl.ds(dst, _RS)] = vs[0]

        for k in range(1, _MAX_PARENTS + 1):
            n_iters = meta_ref[0, 0, k - 1]
            off = meta_ref[0, 0, _MAX_PARENTS + k - 1]
            stride = k + 1
            u = _unroll_for(k)

            def body(i, carry, k=k, off=off, stride=stride, u=u):
                base = off + i * (u * stride)
                for r in range(u):
                    _one(base + r * stride, k)
                return carry

            jax.lax.fori_loop(0, n_iters, body, 0)

        pv = pv_ref[pl.ds(0, _NPD * _RS)]
        emb = emb_ref[0]
        e1 = jnp.dot(emb, w1b_ref[...], preferred_element_type=jnp.float32)
        e1 = jnp.broadcast_to(e1[:, None, :], (_NPD, _B, _HIDDEN))
        e1 = e1.reshape(_NPD * _B, _HIDDEN)
        h = jnp.maximum(
            jnp.dot(pv, w1a_ref[...], preferred_element_type=jnp.float32)
            + e1 + b1_ref[...], 0.0)
        y = (jnp.dot(h, w2_ref[...], preferred_element_type=jnp.float32)
             + b2_ref[...] + pv)
        base = (2 + d * _NPD) * _RS
        v_ref[pl.ds(base, _NPD * _RS)] = y


def kernel(embedding, emb_table, W1, b1, W2, b2):
    meta = jnp.asarray(_META)
    stream = jnp.asarray(_STREAM)
    emb_sl = jax.lax.slice(emb_table, (2, 0), (_TOTAL + 1, _EMB))
    emb_sl = emb_sl.reshape(_MAX_DEPTH, _NPD, _EMB)
    w1a = W1[:_HIDDEN]
    w1b = W1[_HIDDEN:]
    b1r = b1.reshape(1, _HIDDEN)
    b2r = b2.reshape(1, _HIDDEN)

    out = pl.pallas_call(
        _dag_kernel,
        grid=(_MAX_DEPTH,),
        in_specs=[
            pl.BlockSpec((1, 1, 2 * _MAX_PARENTS), lambda d: (d, 0, 0),
                         memory_space=pltpu.SMEM),
            pl.BlockSpec((1, 1, _STREAM.shape[2]), lambda d: (d, 0, 0),
                         memory_space=pltpu.SMEM),
            pl.BlockSpec((1, _NPD, _EMB), lambda d: (d, 0, 0)),
            pl.BlockSpec((_B, _HIDDEN), lambda d: (0, 0)),
            pl.BlockSpec((_HIDDEN, _HIDDEN), lambda d: (0, 0)),
            pl.BlockSpec((_EMB, _HIDDEN), lambda d: (0, 0)),
            pl.BlockSpec((1, _HIDDEN), lambda d: (0, 0)),
            pl.BlockSpec((_HIDDEN, _HIDDEN), lambda d: (0, 0)),
            pl.BlockSpec((1, _HIDDEN), lambda d: (0, 0)),
        ],
        out_specs=pl.BlockSpec(((_TOTAL + 1) * _RS, _HIDDEN), lambda d: (0, 0)),
        out_shape=jax.ShapeDtypeStruct(((_TOTAL + 1) * _RS, _HIDDEN),
                                       jnp.float32),
        scratch_shapes=[pltpu.VMEM((_PV_ROWS * _RS, _HIDDEN), jnp.float32)],
        compiler_params=pltpu.CompilerParams(
            dimension_semantics=("arbitrary",),
            vmem_limit_bytes=56 * 1024 * 1024,
        ),
    )(meta, stream, emb_sl, embedding, w1a, w1b, b1r, W2, b2r)
    out = out.reshape(_TOTAL + 1, _B, _HIDDEN)
    return jnp.transpose(out[1:], (1, 0, 2))
